# baseline (speedup 1.0000x reference)
import functools

import jax
import jax.numpy as jnp
import numpy as np
from jax import lax
from jax.experimental import pallas as pl
from jax.experimental.pallas import tpu as pltpu
from jax.experimental.pallas import tpu_sc as plsc

NC = 2
NS = 16
NW = NC * NS
L = 16
NB = NW
K = 80
CH = 512
PAD = 2048

_SC_PARAMS = pltpu.CompilerParams(needs_layout_passes=False)
_MESH = plsc.VectorSubcoreMesh(core_axis_name="c", subcore_axis_name="s")


def _wid():
    return lax.axis_index("s") * NC + lax.axis_index("c")


def _make_bucket_kernel(E, N, BKT, EP):
    epw = E // NW
    nvec = epw // L

    @functools.partial(
        pl.kernel,
        mesh=_MESH,
        out_type=[
            jax.ShapeDtypeStruct((NW * EP + PAD,), jnp.int32),
            jax.ShapeDtypeStruct((NW * EP + PAD,), jnp.float32),
            jax.ShapeDtypeStruct((NW * EP + PAD,), jnp.int32),
            jax.ShapeDtypeStruct((NW, NB), jnp.int32),
            jax.ShapeDtypeStruct((NW, NB), jnp.int32),
            jax.ShapeDtypeStruct((NW, N), jnp.float32),
        ],
        compiler_params=_SC_PARAMS,
        scratch_types=[
            pltpu.VMEM((epw,), jnp.int32),
            pltpu.VMEM((epw,), jnp.int32),
            pltpu.VMEM((epw,), jnp.float32),
            pltpu.VMEM((epw,), jnp.int32),
            pltpu.VMEM((EP,), jnp.int32),
            pltpu.VMEM((EP,), jnp.float32),
            pltpu.VMEM((EP,), jnp.int32),
            pltpu.VMEM((NB,), jnp.int32),
            pltpu.VMEM((NB,), jnp.int32),
            pltpu.VMEM((NB,), jnp.int32),
            pltpu.VMEM((N,), jnp.float32),
        ],
    )
    def bucket_kernel(src_hbm, dst_hbm, ew_hbm,
                      osrc_hbm, oew_hbm, odst_hbm, ostart_hbm, ocnt_hbm,
                      odeg_hbm,
                      srcb, dstb, ewb, bktb, osrc, oew, odst,
                      hist, cnt, stv, dacc):
        wid = _wid()
        base = wid * epw
        zero_i = jnp.zeros((L,), jnp.int32)
        zero_f = jnp.zeros((L,), jnp.float32)
        ones_i = jnp.ones((L,), jnp.int32)

        pltpu.sync_copy(src_hbm.at[pl.ds(base, epw)], srcb)
        pltpu.sync_copy(dst_hbm.at[pl.ds(base, epw)], dstb)
        pltpu.sync_copy(ew_hbm.at[pl.ds(base, epw)], ewb)

        hist[pl.ds(0, L)] = zero_i
        hist[pl.ds(L, L)] = zero_i

        def dzero(i, carry):
            dacc[pl.ds(i * L, L)] = zero_f
            return carry

        lax.fori_loop(0, N // L, dzero, 0)

        def count_body(i, carry):
            sl = pl.ds(i * L, L)
            d16 = dstb[sl]
            b16 = d16 // BKT
            bktb[sl] = b16
            plsc.addupdate_scatter(hist, [b16], ones_i)
            plsc.addupdate_scatter(dacc, [d16], ewb[sl])
            return carry

        lax.fori_loop(0, nvec, count_body, 0)

        h0 = hist[pl.ds(0, L)]
        h1 = hist[pl.ds(L, L)]
        cp0 = (h0 + (L - 1)) & jnp.full((L,), -L, jnp.int32)
        cp1 = (h1 + (L - 1)) & jnp.full((L,), -L, jnp.int32)
        c0 = plsc.cumsum(cp0)
        c1 = plsc.cumsum(cp1) + jnp.full((L,), c0[L - 1], jnp.int32)
        s0 = c0 - cp0
        s1 = c1 - cp1
        cnt[pl.ds(0, L)] = s0
        cnt[pl.ds(L, L)] = s1
        stv[pl.ds(0, L)] = s0
        stv[pl.ds(L, L)] = s1
        hist[pl.ds(0, L)] = cp0
        hist[pl.ds(L, L)] = cp1

        def zfill(i, carry):
            sl = pl.ds(i * L, L)
            osrc[sl] = zero_i
            oew[sl] = zero_f
            return carry

        lax.fori_loop(0, EP // L, zfill, 0)
        for b in range(NB):
            sb = (s0 if b < L else s1)[b % L]
            cb = (cp0 if b < L else cp1)[b % L]
            dval = jnp.full((L,), b * BKT, jnp.int32)

            def dfill(i, carry):
                odst[pl.ds(pl.multiple_of(sb + i * L, L), L)] = dval
                return carry

            lax.fori_loop(0, cb // L, dfill, 0)

        def place_body(i, carry):
            sl = pl.ds(i * L, L)
            b16 = bktb[sl]
            base16 = plsc.load_gather(cnt, [b16])
            rank, last = plsc.scan_count(b16)
            pos = base16 + rank - ones_i
            plsc.addupdate_scatter(cnt, [b16], rank, mask=last)
            plsc.store_scatter(osrc, [pos], srcb[sl])
            plsc.store_scatter(oew, [pos], ewb[sl])
            plsc.store_scatter(odst, [pos], dstb[sl])
            return carry

        lax.fori_loop(0, nvec, place_body, 0)

        obase = wid * EP
        pltpu.sync_copy(osrc, osrc_hbm.at[pl.ds(obase, EP)])
        pltpu.sync_copy(oew, oew_hbm.at[pl.ds(obase, EP)])
        pltpu.sync_copy(odst, odst_hbm.at[pl.ds(obase, EP)])
        pltpu.sync_copy(stv, ostart_hbm.at[wid])
        pltpu.sync_copy(hist, ocnt_hbm.at[wid])
        pltpu.sync_copy(dacc, odeg_hbm.at[wid])

    return bucket_kernel


def _make_agg_kernel(BKT, EP, NPAD, H, SEG):
    acc_len = BKT * H

    @functools.partial(
        pl.kernel,
        mesh=_MESH,
        out_type=jax.ShapeDtypeStruct((NPAD * H,), jnp.float32),
        compiler_params=_SC_PARAMS,
        scratch_types=[
            pltpu.VMEM((acc_len,), jnp.float32),
            pltpu.VMEM((NW * NB + L,), jnp.int32),
            pltpu.VMEM((NW * NB + L,), jnp.int32),
            pltpu.VMEM((SEG + L,), jnp.int32),
            pltpu.VMEM((SEG + L,), jnp.int32),
            pltpu.VMEM((SEG + L,), jnp.float32),
            pltpu.VMEM((SEG + L,), jnp.int32),
            pltpu.VMEM((SEG + L,), jnp.int32),
            pltpu.VMEM((SEG + L,), jnp.float32),
            pltpu.VMEM((K, H), jnp.float32),
            pltpu.VMEM((K, H), jnp.float32),
            pltpu.VMEM((L, H), jnp.float32),
            pltpu.SemaphoreType.DMA,
            pltpu.SemaphoreType.DMA,
            pltpu.SemaphoreType.DMA,
            pltpu.SemaphoreType.DMA,
            pltpu.SemaphoreType.DMA,
        ],
    )
    def agg_kernel(g_hbm, src_hbm, ew_hbm, dst_hbm, start_hbm, cnt_hbm,
                   out_hbm, acc, stv, cpv,
                   ss0, sd0, sw0, ss1, sd1, sw1, r0, r1, rt,
                   qs0, qs1, qr0, qr1, qrt):
        b = _wid()
        nodebase = b * BKT
        zero = jnp.zeros((L,), jnp.float32)
        nb16 = jnp.full((L,), nodebase, jnp.int32)
        iota16 = lax.iota(jnp.int32, L)

        pltpu.sync_copy(start_hbm, stv.at[pl.ds(0, NW * NB)])
        pltpu.sync_copy(cnt_hbm, cpv.at[pl.ds(0, NW * NB)])

        def seg_meta(w):
            off = w * NB + b
            s = stv[pl.ds(off, L)][0]
            c = cpv[pl.ds(off, L)][0]
            return w * EP + pl.multiple_of(s, L), c

        def fire_stage(sb_, db_, wb_, sem, eb, c):
            nsub = (c + CH - 1) // CH

            def sub(r, carry):
                hoff = pl.multiple_of(eb + r * CH, L)
                voff = pl.multiple_of(r * CH, L)
                pltpu.async_copy(
                    src_hbm.at[pl.ds(hoff, CH)], sb_.at[pl.ds(voff, CH)], sem)
                pltpu.async_copy(
                    dst_hbm.at[pl.ds(hoff, CH)], db_.at[pl.ds(voff, CH)], sem)
                pltpu.async_copy(
                    ew_hbm.at[pl.ds(hoff, CH)], wb_.at[pl.ds(voff, CH)], sem)
                return carry

            lax.fori_loop(0, nsub, sub, 0)

        def wait_stage(sb_, db_, wb_, sem, c):
            nsub = (c + CH - 1) // CH

            def sub(r, carry):
                voff = pl.multiple_of(r * CH, L)
                pltpu.make_async_copy(
                    src_hbm.at[pl.ds(0, CH)], sb_.at[pl.ds(voff, CH)],
                    sem).wait()
                pltpu.make_async_copy(
                    dst_hbm.at[pl.ds(0, CH)], db_.at[pl.ds(voff, CH)],
                    sem).wait()
                pltpu.make_async_copy(
                    ew_hbm.at[pl.ds(0, CH)], wb_.at[pl.ds(voff, CH)],
                    sem).wait()
                return carry

            lax.fori_loop(0, nsub, sub, 0)

        def fire_gather(sb_, rows_ref, sem, toff, ch):
            idx = sb_.at[pl.ds(pl.multiple_of(toff, L), ch)]
            pltpu.async_copy(g_hbm.at[idx], rows_ref, sem)

        def wait_gather(rows_ref, sem, ch):
            pltpu.make_async_copy(
                g_hbm.at[pl.ds(0, ch)], rows_ref, sem).wait()

        def compute_chunk(rows_ref, db_, wb_, toff, ch):
            @plsc.parallel_loop(0, ch, 1, unroll=8)
            def _(j):
                ewj = jnp.full((L,), wb_[pl.ds(toff + j, L)][0], jnp.float32)
                rowoff = (db_[pl.ds(toff + j, L)][0] - nodebase) * H
                for v in range(H // L):
                    plsc.addupdate(
                        acc.at[pl.ds(rowoff + v * L, L)],
                        rows_ref[j, pl.ds(v * L, L)] * ewj)

        def process_seg(sb_, db_, wb_, c):
            nfull = c // K
            ntail = (c % K) // L

            @pl.when(nfull > 0)
            def _():
                fire_gather(sb_, r0, qr0, 0, K)

            def pair(pi, carry):
                t0 = pi * 2

                @pl.when(t0 + 1 < nfull)
                def _():
                    fire_gather(sb_, r1, qr1, (t0 + 1) * K, K)

                wait_gather(r0, qr0, K)
                compute_chunk(r0, db_, wb_, t0 * K, K)

                @pl.when(t0 + 2 < nfull)
                def _():
                    fire_gather(sb_, r0, qr0, (t0 + 2) * K, K)

                @pl.when(t0 + 1 < nfull)
                def _():
                    wait_gather(r1, qr1, K)
                    compute_chunk(r1, db_, wb_, (t0 + 1) * K, K)

                return carry

            lax.fori_loop(0, (nfull + 1) // 2, pair, 0)

            tb = nfull * K

            def tl(t, carry):
                fire_gather(sb_, rt, qrt, tb + t * L, L)
                wait_gather(rt, qrt, L)
                compute_chunk(rt, db_, wb_, tb + t * L, L)
                return carry

            lax.fori_loop(0, ntail, tl, 0)

        eb0, c0 = seg_meta(0)
        fire_stage(ss0, sd0, sw0, qs0, eb0, c0)

        def zbody(i, carry):
            acc[pl.ds(i * L, L)] = zero
            return carry

        lax.fori_loop(0, acc_len // L, zbody, 0)

        def segpair(p, carry):
            eb_e, c_e = carry
            eb_o, c_o = seg_meta(2 * p + 1)
            fire_stage(ss1, sd1, sw1, qs1, eb_o, c_o)
            wait_stage(ss0, sd0, sw0, qs0, c_e)
            process_seg(ss0, sd0, sw0, c_e)

            w_ne = 2 * p + 2
            eb_ne, c_ne = seg_meta(jnp.minimum(w_ne, NW - 1))

            @pl.when(w_ne < NW)
            def _():
                fire_stage(ss0, sd0, sw0, qs0, eb_ne, c_ne)

            wait_stage(ss1, sd1, sw1, qs1, c_o)
            process_seg(ss1, sd1, sw1, c_o)
            return (eb_ne, c_ne)

        lax.fori_loop(0, NW // 2, segpair, (eb0, c0))

        pltpu.sync_copy(
            acc, out_hbm.at[pl.ds(pl.multiple_of(nodebase * H, 8), acc_len)])

    return agg_kernel


def _tc_ab_body(x_ref, w1_ref, b1_ref, degp_ref, wc1_ref, bc1_ref,
                g1_ref, e1_ref):
    deg = jnp.sum(degp_ref[...], axis=0) + 1.0
    dis = lax.rsqrt(deg)
    inv = 1.0 / deg
    h1 = jnp.maximum(
        jnp.dot(x_ref[...], w1_ref[...], preferred_element_type=jnp.float32)
        + b1_ref[...], 0.0)
    t = jnp.dot(h1, wc1_ref[...], preferred_element_type=jnp.float32)
    g1_ref[...] = dis[:, None] * t
    e1_ref[...] = inv[:, None] * t + bc1_ref[...]


def _tc_mid_body(s_ref, e_ref, degp_ref, w_ref, b_ref, g_ref, e2_ref):
    deg = jnp.sum(degp_ref[...], axis=0) + 1.0
    dis = lax.rsqrt(deg)
    inv = 1.0 / deg
    h = jnp.maximum(dis[:, None] * s_ref[...] + e_ref[...], 0.0)
    t = jnp.dot(h, w_ref[...], preferred_element_type=jnp.float32)
    g_ref[...] = dis[:, None] * t
    e2_ref[...] = inv[:, None] * t + b_ref[...]


def _tc_final_body(s_ref, e_ref, degp_ref, w_ref, b_ref, out_ref):
    deg = jnp.sum(degp_ref[...], axis=0) + 1.0
    dis = lax.rsqrt(deg)
    h = jnp.maximum(dis[:, None] * s_ref[...] + e_ref[...], 0.0)
    out_ref[...] = (
        jnp.dot(h, w_ref[...], preferred_element_type=jnp.float32)
        + b_ref[...])


def kernel(x, edge_index, edge_weight, W_lin1, b_lin1, W_conv1, b_conv1,
           W_conv2, b_conv2, W_lin2, b_lin2):
    N, F = x.shape
    H = W_conv1.shape[0]
    C = W_lin2.shape[1]
    E = edge_weight.shape[0]
    BN = 1024
    NPAD = ((N + BN - 1) // BN) * BN
    grid = NPAD // BN
    BKT = NPAD // NB
    epw = E // NW
    EP = epw + NB * L
    SEG = epw

    src = edge_index[0]
    dst = edge_index[1]

    bucket_fn = _make_bucket_kernel(E, N, BKT, EP)
    src_s, ew_s, dst_s, starts, cnts, degp = bucket_fn(src, dst, edge_weight)
    starts = starts.reshape(NW * NB)
    cnts = cnts.reshape(NW * NB)
    degp = jnp.pad(degp, ((0, 0), (0, NPAD - N)))

    xp = jnp.pad(x, ((0, NPAD - N), (0, 0)))
    b1r = b_lin1.reshape(1, F)
    bc1r = b_conv1.reshape(1, H)
    bc2r = b_conv2.reshape(1, H)
    w2p = jnp.zeros((H, 128), jnp.float32).at[:, :C].set(W_lin2)
    b2p = jnp.zeros((1, 128), jnp.float32).at[0, :C].set(b_lin2)

    full = lambda shape: pl.BlockSpec(shape, lambda i: (0,) * len(shape))
    row = pl.BlockSpec((BN, H), lambda i: (i, 0))
    degp_spec = pl.BlockSpec((NW, BN), lambda i: (0, i))
    g_shape = jax.ShapeDtypeStruct((NPAD, H), jnp.float32)
    e_shape = jax.ShapeDtypeStruct((NPAD, H), jnp.float32)

    g1, e1 = pl.pallas_call(
        _tc_ab_body,
        grid=(grid,),
        in_specs=[row, full((F, H)), full((1, H)), degp_spec,
                  full((H, H)), full((1, H))],
        out_specs=[row, row],
        out_shape=[g_shape, e_shape],
    )(xp, W_lin1, b1r, degp, W_conv1, bc1r)

    agg_fn = _make_agg_kernel(BKT, EP, NPAD, H, SEG)
    s1 = agg_fn(g1, src_s, ew_s, dst_s, starts, cnts).reshape(NPAD, H)

    g2, e2 = pl.pallas_call(
        _tc_mid_body,
        grid=(grid,),
        in_specs=[row, row, degp_spec, full((H, H)), full((1, H))],
        out_specs=[row, row],
        out_shape=[g_shape, e_shape],
    )(s1, e1, degp, W_conv2, bc2r)

    s2 = agg_fn(g2, src_s, ew_s, dst_s, starts, cnts).reshape(NPAD, H)

    outp = pl.pallas_call(
        _tc_final_body,
        grid=(grid,),
        in_specs=[row, row, degp_spec, full((H, 128)), full((1, 128))],
        out_specs=pl.BlockSpec((BN, 128), lambda i: (i, 0)),
        out_shape=jax.ShapeDtypeStruct((NPAD, 128), jnp.float32),
    )(s2, e2, degp, w2p, b2p)

    return outp[:N, :C]


# speedup vs baseline: 1.0005x; 1.0005x over previous
import functools

import jax
import jax.numpy as jnp
from jax import lax
from jax.experimental import pallas as pl
from jax.experimental.pallas import tpu as pltpu
from jax.experimental.pallas import tpu_sc as plsc

NC = 2
NS = 16
NW = NC * NS
L = 16
NB = NW
K = 80
CH = 512
PAD = 2048

_SC_PARAMS = pltpu.CompilerParams(needs_layout_passes=False)
_MESH = plsc.VectorSubcoreMesh(core_axis_name="c", subcore_axis_name="s")


def _wid():
    return lax.axis_index("s") * NC + lax.axis_index("c")


def _make_bucket_kernel(E, N, BKT, EP):
    epw = E // NW
    nvec = epw // L

    @functools.partial(
        pl.kernel,
        mesh=_MESH,
        out_type=[
            jax.ShapeDtypeStruct((NW * EP + PAD,), jnp.int32),
            jax.ShapeDtypeStruct((NW * EP + PAD,), jnp.float32),
            jax.ShapeDtypeStruct((NW * EP + PAD,), jnp.int32),
            jax.ShapeDtypeStruct((NW, NB), jnp.int32),
            jax.ShapeDtypeStruct((NW, NB), jnp.int32),
            jax.ShapeDtypeStruct((NW, N), jnp.float32),
        ],
        compiler_params=_SC_PARAMS,
        scratch_types=[
            pltpu.VMEM((epw,), jnp.int32),
            pltpu.VMEM((epw,), jnp.int32),
            pltpu.VMEM((epw,), jnp.float32),
            pltpu.VMEM((epw,), jnp.int32),
            pltpu.VMEM((EP,), jnp.int32),
            pltpu.VMEM((EP,), jnp.float32),
            pltpu.VMEM((EP,), jnp.int32),
            pltpu.VMEM((NB,), jnp.int32),
            pltpu.VMEM((NB,), jnp.int32),
            pltpu.VMEM((NB,), jnp.int32),
            pltpu.VMEM((N,), jnp.float32),
        ],
    )
    def bucket_kernel(src_hbm, dst_hbm, ew_hbm,
                      osrc_hbm, oew_hbm, odst_hbm, ostart_hbm, ocnt_hbm,
                      odeg_hbm,
                      srcb, dstb, ewb, bktb, osrc, oew, odst,
                      hist, cnt, stv, dacc):
        wid = _wid()
        base = wid * epw
        zero_i = jnp.zeros((L,), jnp.int32)
        zero_f = jnp.zeros((L,), jnp.float32)
        ones_i = jnp.ones((L,), jnp.int32)

        pltpu.sync_copy(src_hbm.at[pl.ds(base, epw)], srcb)
        pltpu.sync_copy(dst_hbm.at[pl.ds(base, epw)], dstb)
        pltpu.sync_copy(ew_hbm.at[pl.ds(base, epw)], ewb)

        hist[pl.ds(0, L)] = zero_i
        hist[pl.ds(L, L)] = zero_i

        def dzero(i, carry):
            dacc[pl.ds(i * L, L)] = zero_f
            return carry

        lax.fori_loop(0, N // L, dzero, 0)

        def count_body(i, carry):
            sl = pl.ds(i * L, L)
            d16 = dstb[sl]
            b16 = d16 // BKT
            bktb[sl] = b16
            plsc.addupdate_scatter(hist, [b16], ones_i)
            plsc.addupdate_scatter(dacc, [d16], ewb[sl])
            return carry

        lax.fori_loop(0, nvec, count_body, 0)

        h0 = hist[pl.ds(0, L)]
        h1 = hist[pl.ds(L, L)]
        cp0 = (h0 + (L - 1)) & jnp.full((L,), -L, jnp.int32)
        cp1 = (h1 + (L - 1)) & jnp.full((L,), -L, jnp.int32)
        c0 = plsc.cumsum(cp0)
        c1 = plsc.cumsum(cp1) + jnp.full((L,), c0[L - 1], jnp.int32)
        s0 = c0 - cp0
        s1 = c1 - cp1
        cnt[pl.ds(0, L)] = s0
        cnt[pl.ds(L, L)] = s1
        stv[pl.ds(0, L)] = s0
        stv[pl.ds(L, L)] = s1
        hist[pl.ds(0, L)] = cp0
        hist[pl.ds(L, L)] = cp1

        def zfill(i, carry):
            sl = pl.ds(i * L, L)
            osrc[sl] = zero_i
            oew[sl] = zero_f
            return carry

        lax.fori_loop(0, EP // L, zfill, 0)
        for b in range(NB):
            sb = (s0 if b < L else s1)[b % L]
            cb = (cp0 if b < L else cp1)[b % L]
            dval = jnp.full((L,), b * BKT, jnp.int32)

            def dfill(i, carry):
                odst[pl.ds(pl.multiple_of(sb + i * L, L), L)] = dval
                return carry

            lax.fori_loop(0, cb // L, dfill, 0)

        def place_body(i, carry):
            sl = pl.ds(i * L, L)
            b16 = bktb[sl]
            base16 = plsc.load_gather(cnt, [b16])
            rank, last = plsc.scan_count(b16)
            pos = base16 + rank - ones_i
            plsc.addupdate_scatter(cnt, [b16], rank, mask=last)
            plsc.store_scatter(osrc, [pos], srcb[sl])
            plsc.store_scatter(oew, [pos], ewb[sl])
            plsc.store_scatter(odst, [pos], dstb[sl])
            return carry

        lax.fori_loop(0, nvec, place_body, 0)

        obase = wid * EP
        pltpu.sync_copy(osrc, osrc_hbm.at[pl.ds(obase, EP)])
        pltpu.sync_copy(oew, oew_hbm.at[pl.ds(obase, EP)])
        pltpu.sync_copy(odst, odst_hbm.at[pl.ds(obase, EP)])
        pltpu.sync_copy(stv, ostart_hbm.at[wid])
        pltpu.sync_copy(hist, ocnt_hbm.at[wid])
        pltpu.sync_copy(dacc, odeg_hbm.at[wid])

    return bucket_kernel


def _make_agg_kernel(BKT, EP, NPAD, H, SEG):
    acc_len = BKT * H

    @functools.partial(
        pl.kernel,
        mesh=_MESH,
        out_type=jax.ShapeDtypeStruct((NPAD * H,), jnp.float32),
        compiler_params=_SC_PARAMS,
        scratch_types=[
            pltpu.VMEM((acc_len,), jnp.float32),
            pltpu.VMEM((NW * NB + L,), jnp.int32),
            pltpu.VMEM((NW * NB + L,), jnp.int32),
            pltpu.VMEM((SEG + L,), jnp.int32),
            pltpu.VMEM((SEG + L,), jnp.int32),
            pltpu.VMEM((SEG + L,), jnp.float32),
            pltpu.VMEM((SEG + L,), jnp.int32),
            pltpu.VMEM((SEG + L,), jnp.int32),
            pltpu.VMEM((SEG + L,), jnp.float32),
            pltpu.VMEM((K, H), jnp.float32),
            pltpu.VMEM((K, H), jnp.float32),
            pltpu.VMEM((L, H), jnp.float32),
            pltpu.SemaphoreType.DMA,
            pltpu.SemaphoreType.DMA,
            pltpu.SemaphoreType.DMA,
            pltpu.SemaphoreType.DMA,
            pltpu.SemaphoreType.DMA,
        ],
    )
    def agg_kernel(g_hbm, src_hbm, ew_hbm, dst_hbm, start_hbm, cnt_hbm,
                   out_hbm, acc, stv, cpv,
                   ss0, sd0, sw0, ss1, sd1, sw1, r0, r1, rt,
                   qs0, qs1, qr0, qr1, qrt):
        b = _wid()
        nodebase = b * BKT
        zero = jnp.zeros((L,), jnp.float32)

        pltpu.sync_copy(start_hbm, stv.at[pl.ds(0, NW * NB)])
        pltpu.sync_copy(cnt_hbm, cpv.at[pl.ds(0, NW * NB)])

        def seg_meta(w):
            off = w * NB + b
            s = stv[pl.ds(off, L)][0]
            c = cpv[pl.ds(off, L)][0]
            return w * EP + pl.multiple_of(s, L), c

        def fire_stage(sb_, db_, wb_, sem, eb, c):
            nsub = (c + CH - 1) // CH

            def sub(r, carry):
                hoff = pl.multiple_of(eb + r * CH, L)
                voff = pl.multiple_of(r * CH, L)
                pltpu.async_copy(
                    src_hbm.at[pl.ds(hoff, CH)], sb_.at[pl.ds(voff, CH)], sem)
                pltpu.async_copy(
                    dst_hbm.at[pl.ds(hoff, CH)], db_.at[pl.ds(voff, CH)], sem)
                pltpu.async_copy(
                    ew_hbm.at[pl.ds(hoff, CH)], wb_.at[pl.ds(voff, CH)], sem)
                return carry

            lax.fori_loop(0, nsub, sub, 0)

        def wait_stage(sb_, db_, wb_, sem, c):
            nsub = (c + CH - 1) // CH

            def sub(r, carry):
                voff = pl.multiple_of(r * CH, L)
                pltpu.make_async_copy(
                    src_hbm.at[pl.ds(0, CH)], sb_.at[pl.ds(voff, CH)],
                    sem).wait()
                pltpu.make_async_copy(
                    dst_hbm.at[pl.ds(0, CH)], db_.at[pl.ds(voff, CH)],
                    sem).wait()
                pltpu.make_async_copy(
                    ew_hbm.at[pl.ds(0, CH)], wb_.at[pl.ds(voff, CH)],
                    sem).wait()
                return carry

            lax.fori_loop(0, nsub, sub, 0)

        def fire_gather(sb_, rows_ref, sem, toff, ch):
            idx = sb_.at[pl.ds(pl.multiple_of(toff, L), ch)]
            pltpu.async_copy(g_hbm.at[idx], rows_ref, sem)

        def wait_gather(rows_ref, sem, ch):
            pltpu.make_async_copy(
                g_hbm.at[pl.ds(0, ch)], rows_ref, sem).wait()

        def compute_chunk(rows_ref, db_, wb_, toff, ch):
            @plsc.parallel_loop(0, ch, 1, unroll=8)
            def _(j):
                ewj = jnp.full((L,), wb_[pl.ds(toff + j, L)][0], jnp.float32)
                rowoff = (db_[pl.ds(toff + j, L)][0] - nodebase) * H
                for v in range(H // L):
                    plsc.addupdate(
                        acc.at[pl.ds(rowoff + v * L, L)],
                        rows_ref[j, pl.ds(v * L, L)] * ewj)

        def process_seg(sb_, db_, wb_, c):
            nfull = c // K
            ntail = (c % K) // L

            @pl.when(nfull > 0)
            def _():
                fire_gather(sb_, r0, qr0, 0, K)

            def pair(pi, carry):
                t0 = pi * 2

                @pl.when(t0 + 1 < nfull)
                def _():
                    fire_gather(sb_, r1, qr1, (t0 + 1) * K, K)

                wait_gather(r0, qr0, K)
                compute_chunk(r0, db_, wb_, t0 * K, K)

                @pl.when(t0 + 2 < nfull)
                def _():
                    fire_gather(sb_, r0, qr0, (t0 + 2) * K, K)

                @pl.when(t0 + 1 < nfull)
                def _():
                    wait_gather(r1, qr1, K)
                    compute_chunk(r1, db_, wb_, (t0 + 1) * K, K)

                return carry

            lax.fori_loop(0, (nfull + 1) // 2, pair, 0)

            tb = nfull * K

            def tl(t, carry):
                fire_gather(sb_, rt, qrt, tb + t * L, L)
                wait_gather(rt, qrt, L)
                compute_chunk(rt, db_, wb_, tb + t * L, L)
                return carry

            lax.fori_loop(0, ntail, tl, 0)

        eb0, c0 = seg_meta(0)
        fire_stage(ss0, sd0, sw0, qs0, eb0, c0)

        def zbody(i, carry):
            acc[pl.ds(i * L, L)] = zero
            return carry

        lax.fori_loop(0, acc_len // L, zbody, 0)

        def segpair(p, carry):
            eb_e, c_e = carry
            eb_o, c_o = seg_meta(2 * p + 1)
            fire_stage(ss1, sd1, sw1, qs1, eb_o, c_o)
            wait_stage(ss0, sd0, sw0, qs0, c_e)
            process_seg(ss0, sd0, sw0, c_e)

            w_ne = 2 * p + 2
            eb_ne, c_ne = seg_meta(jnp.minimum(w_ne, NW - 1))

            @pl.when(w_ne < NW)
            def _():
                fire_stage(ss0, sd0, sw0, qs0, eb_ne, c_ne)

            wait_stage(ss1, sd1, sw1, qs1, c_o)
            process_seg(ss1, sd1, sw1, c_o)
            return (eb_ne, c_ne)

        lax.fori_loop(0, NW // 2, segpair, (eb0, c0))

        pltpu.sync_copy(
            acc, out_hbm.at[pl.ds(pl.multiple_of(nodebase * H, 8), acc_len)])

    return agg_kernel


def _tc_ab_body(x_ref, w1_ref, b1_ref, degp_ref, wc1_ref, bc1_ref,
                g1_ref, e1_ref):
    deg = jnp.sum(degp_ref[...], axis=0) + 1.0
    dis = lax.rsqrt(deg)
    inv = 1.0 / deg
    h1 = jnp.maximum(
        jnp.dot(x_ref[...], w1_ref[...], preferred_element_type=jnp.float32)
        + b1_ref[...], 0.0)
    t = jnp.dot(h1, wc1_ref[...], preferred_element_type=jnp.float32)
    g1_ref[...] = dis[:, None] * t
    e1_ref[...] = inv[:, None] * t + bc1_ref[...]


def _tc_mid_body(s_ref, e_ref, degp_ref, w_ref, b_ref, g_ref, e2_ref):
    deg = jnp.sum(degp_ref[...], axis=0) + 1.0
    dis = lax.rsqrt(deg)
    inv = 1.0 / deg
    h = jnp.maximum(dis[:, None] * s_ref[...] + e_ref[...], 0.0)
    t = jnp.dot(h, w_ref[...], preferred_element_type=jnp.float32)
    g_ref[...] = dis[:, None] * t
    e2_ref[...] = inv[:, None] * t + b_ref[...]


def _tc_final_body(s_ref, e_ref, degp_ref, w_ref, b_ref, out_ref):
    deg = jnp.sum(degp_ref[...], axis=0) + 1.0
    dis = lax.rsqrt(deg)
    h = jnp.maximum(dis[:, None] * s_ref[...] + e_ref[...], 0.0)
    out_ref[...] = (
        jnp.dot(h, w_ref[...], preferred_element_type=jnp.float32)
        + b_ref[...])


def kernel(x, edge_index, edge_weight, W_lin1, b_lin1, W_conv1, b_conv1,
           W_conv2, b_conv2, W_lin2, b_lin2):
    N, F = x.shape
    H = W_conv1.shape[0]
    C = W_lin2.shape[1]
    E = edge_weight.shape[0]
    BN = 1024
    NPAD = ((N + BN - 1) // BN) * BN
    grid = NPAD // BN
    BKT = NPAD // NB
    epw = E // NW
    EP = epw + NB * L
    SEG = epw

    src = edge_index[0]
    dst = edge_index[1]

    bucket_fn = _make_bucket_kernel(E, N, BKT, EP)
    src_s, ew_s, dst_s, starts, cnts, degp = bucket_fn(src, dst, edge_weight)
    starts = starts.reshape(NW * NB)
    cnts = cnts.reshape(NW * NB)
    degp = jnp.pad(degp, ((0, 0), (0, NPAD - N)))

    xp = jnp.pad(x, ((0, NPAD - N), (0, 0)))
    b1r = b_lin1.reshape(1, F)
    bc1r = b_conv1.reshape(1, H)
    bc2r = b_conv2.reshape(1, H)
    w2p = jnp.zeros((H, 128), jnp.float32).at[:, :C].set(W_lin2)
    b2p = jnp.zeros((1, 128), jnp.float32).at[0, :C].set(b_lin2)

    full = lambda shape: pl.BlockSpec(shape, lambda i: (0,) * len(shape))
    row = pl.BlockSpec((BN, H), lambda i: (i, 0))
    degp_spec = pl.BlockSpec((NW, BN), lambda i: (0, i))
    g_shape = jax.ShapeDtypeStruct((NPAD, H), jnp.float32)
    e_shape = jax.ShapeDtypeStruct((NPAD, H), jnp.float32)

    g1, e1 = pl.pallas_call(
        _tc_ab_body,
        grid=(grid,),
        in_specs=[row, full((F, H)), full((1, H)), degp_spec,
                  full((H, H)), full((1, H))],
        out_specs=[row, row],
        out_shape=[g_shape, e_shape],
    )(xp, W_lin1, b1r, degp, W_conv1, bc1r)

    agg_fn = _make_agg_kernel(BKT, EP, NPAD, H, SEG)
    s1 = agg_fn(g1, src_s, ew_s, dst_s, starts, cnts).reshape(NPAD, H)

    g2, e2 = pl.pallas_call(
        _tc_mid_body,
        grid=(grid,),
        in_specs=[row, row, degp_spec, full((H, H)), full((1, H))],
        out_specs=[row, row],
        out_shape=[g_shape, e_shape],
    )(s1, e1, degp, W_conv2, bc2r)

    s2 = agg_fn(g2, src_s, ew_s, dst_s, starts, cnts).reshape(NPAD, H)

    outp = pl.pallas_call(
        _tc_final_body,
        grid=(grid,),
        in_specs=[row, row, degp_spec, full((H, 128)), full((1, 128))],
        out_specs=pl.BlockSpec((BN, 128), lambda i: (i, 0)),
        out_shape=jax.ShapeDtypeStruct((NPAD, 128), jnp.float32),
    )(s2, e2, degp, w2p, b2p)

    return outp[:N, :C]
